# baseline (speedup 1.0000x reference)
import jax
import jax.numpy as jnp
from jax.experimental import pallas as pl

NF = 9


def kernel(x, emb1, emb2_first, emb2_second, emb3, title_table, Ws, bs, bn_scales, bn_biases):
    feats = []
    for i in range(NF):
        feats.append(jnp.take(emb1[i], x[:, i], axis=0))
    inc = 0
    for i in range(NF):
        for j in range(i, NF):
            a = jnp.take(emb2_first[inc], x[:, i], axis=0)
            b = jnp.take(emb2_second[inc], x[:, j], axis=0)
            feats.append(a * b)
            inc += 1
    for i in range(9):
        feats.append(jnp.take(emb3[i], x[:, i + 9], axis=0))
    feats.append(jnp.sum(jnp.take(title_table, x[:, 18:28], axis=0), axis=1))
    feats.append(x[:, 28:156].astype(jnp.float32))
    feats.append(x[:, 156:].astype(jnp.float32))
    out = jnp.concatenate(feats, axis=1)
    for k in range(len(Ws)):
        out = out @ Ws[k] + bs[k]
        if k != len(Ws) - 1:
            m = jnp.mean(out, axis=0)
            v = jnp.var(out, axis=0)
            out = (out - m) / jnp.sqrt(v + 1e-5) * bn_scales[k] + bn_biases[k]
    return jax.nn.sigmoid(out)


# speedup vs baseline: 2.6457x; 2.6457x over previous
import functools

import jax
import jax.numpy as jnp
from jax import lax
from jax.experimental import pallas as pl
from jax.experimental.pallas import tpu as pltpu
from jax.experimental.pallas import tpu_sc as plsc

EMB = 128
NF = 9
BATCH = 4096
PAIRS = [(i, j) for i in range(NF) for j in range(i, NF)]
NPAIR = len(PAIRS)
NPLANE = NF + NPAIR + 9 + 1
NC, NS, LANES = 2, 16, 16
NW = NC * NS
BPW = BATCH // NW
D1 = 1024
NDENSE = 256

_MESH = plsc.VectorSubcoreMesh(
    core_axis_name="c", subcore_axis_name="s", num_cores=NC, num_subcores=NS)

_NB = 3


def _sc_body(*refs):
    xt = refs[0]
    e1 = refs[1:1 + NF]
    ef = refs[1 + NF:1 + NF + NPAIR]
    es = refs[1 + NF + NPAIR:1 + NF + 2 * NPAIR]
    e3 = refs[1 + NF + 2 * NPAIR:1 + NF + 2 * NPAIR + 9]
    tt = refs[1 + NF + 2 * NPAIR + 9]
    g = refs[1 + NF + 2 * NPAIR + 10]
    idx_v, bufa, bufb, acc_v, sem_i, sem_o = refs[1 + NF + 2 * NPAIR + 11:]

    wid = lax.axis_index("s") * NC + lax.axis_index("c")
    base = wid * BPW

    pltpu.sync_copy(xt.at[:, pl.ds(base, BPW)], idx_v)

    jobs = []
    for i in range(NF):
        jobs.append((e1[i], i, None, None, i))
    for p, (i, j) in enumerate(PAIRS):
        jobs.append((ef[p], i, es[p], j, NF + p))
    for i in range(9):
        jobs.append((e3[i], NF + i, None, None, NF + NPAIR + i))

    def issue(jx):
        t, c, t2, c2, _ = jobs[jx]
        slot = jx % _NB
        d1 = pltpu.async_copy(t.at[idx_v.at[c]], bufa.at[slot], sem_i)
        d2 = None
        if t2 is not None:
            d2 = pltpu.async_copy(t2.at[idx_v.at[c2]], bufb.at[slot], sem_i)
        return d1, d2

    pending = {0: issue(0)}
    writes = []
    for jx in range(len(jobs)):
        _, _, t2, _, plane = jobs[jx]
        slot = jx % _NB
        d1, d2 = pending.pop(jx)
        d1.wait()
        if d2 is not None:
            d2.wait()
        if jx + 1 < len(jobs):
            if len(writes) >= _NB - 1:
                writes.pop(0).wait()
            pending[jx + 1] = issue(jx + 1)
        if t2 is not None:
            def mul_body(r, carry, _slot=slot):
                for v in range(EMB // LANES):
                    sl = pl.ds(v * LANES, LANES)
                    bufa[_slot, r, sl] = bufa[_slot, r, sl] * bufb[_slot, r, sl]
                return carry
            lax.fori_loop(0, BPW, mul_body, 0)
        writes.append(pltpu.async_copy(
            bufa.at[slot], g.at[pl.ds(plane * BATCH + base, BPW)], sem_o))
    for w in writes:
        w.wait()

    d_acc = pltpu.async_copy(tt.at[idx_v.at[18]], acc_v, sem_i)
    d_acc.wait()
    d_next = pltpu.async_copy(tt.at[idx_v.at[19]], bufa.at[0], sem_i)
    for k in range(1, 10):
        d_next.wait()
        cur = (k - 1) % 2
        if k < 9:
            d_next = pltpu.async_copy(
                tt.at[idx_v.at[18 + k + 1]], bufa.at[k % 2], sem_i)
        def add_body(r, carry, _cur=cur):
            for v in range(EMB // LANES):
                sl = pl.ds(v * LANES, LANES)
                acc_v[r, sl] = acc_v[r, sl] + bufa[_cur, r, sl]
            return carry
        lax.fori_loop(0, BPW, add_body, 0)
    pltpu.sync_copy(acc_v, g.at[pl.ds((NPLANE - 1) * BATCH + base, BPW)])


_sc_gather = functools.partial(
    pl.kernel,
    out_type=jax.ShapeDtypeStruct((NPLANE * BATCH, EMB), jnp.float32),
    mesh=_MESH,
    scratch_types=[
        pltpu.VMEM((28, BPW), jnp.int32),
        pltpu.VMEM((_NB, BPW, EMB), jnp.float32),
        pltpu.VMEM((_NB, BPW, EMB), jnp.float32),
        pltpu.VMEM((BPW, EMB), jnp.float32),
        pltpu.SemaphoreType.DMA,
        pltpu.SemaphoreType.DMA,
    ],
)(_sc_body)


BM = 1024
BT = BATCH // BM
KP = 8
KT = NPLANE // KP


def _l1_body(g_ref, xd_ref, we_ref, wd_ref, b_ref, y_ref, s_ref, q_ref, acc_ref):
    kt = pl.program_id(1)

    @pl.when(kt == 0)
    def _():
        xd = xd_ref[...].astype(jnp.float32)
        acc_ref[...] = jnp.dot(
            xd, wd_ref[...], preferred_element_type=jnp.float32) + b_ref[...]

    part = None
    for t in range(KP):
        d = jnp.dot(g_ref[t].astype(jnp.bfloat16),
                    we_ref[t].astype(jnp.bfloat16),
                    preferred_element_type=jnp.float32)
        part = d if part is None else part + d
    acc_ref[...] += part

    @pl.when(kt == KT - 1)
    def _():
        y = acc_ref[...]
        y_ref[...] = y
        s_ref[...] = jnp.sum(y, axis=0, keepdims=True)[None]
        q_ref[...] = jnp.sum(y * y, axis=0, keepdims=True)[None]


def _layer1(g3, xd, we, wd, b1):
    return pl.pallas_call(
        _l1_body,
        grid=(BT, KT),
        in_specs=[
            pl.BlockSpec((KP, BM, EMB), lambda i, k: (k, i, 0)),
            pl.BlockSpec((BM, NDENSE), lambda i, k: (i, 0)),
            pl.BlockSpec((KP, EMB, D1), lambda i, k: (k, 0, 0)),
            pl.BlockSpec((NDENSE, D1), lambda i, k: (0, 0)),
            pl.BlockSpec((1, D1), lambda i, k: (0, 0)),
        ],
        out_specs=[
            pl.BlockSpec((BM, D1), lambda i, k: (i, 0)),
            pl.BlockSpec((1, 1, D1), lambda i, k: (i, 0, 0)),
            pl.BlockSpec((1, 1, D1), lambda i, k: (i, 0, 0)),
        ],
        out_shape=[
            jax.ShapeDtypeStruct((BATCH, D1), jnp.float32),
            jax.ShapeDtypeStruct((BT, 1, D1), jnp.float32),
            jax.ShapeDtypeStruct((BT, 1, D1), jnp.float32),
        ],
        scratch_shapes=[pltpu.VMEM((BM, D1), jnp.float32)],
        compiler_params=pltpu.CompilerParams(
            dimension_semantics=("parallel", "arbitrary")),
    )(g3, xd, we, wd, b1)


def _mid_body(y_ref, s_ref, q_ref, w_ref, b_ref, gm_ref, bb_ref,
              y2_ref, s2_ref, q2_ref):
    m = jnp.sum(s_ref[...], axis=0) * (1.0 / BATCH)
    ex2 = jnp.sum(q_ref[...], axis=0) * (1.0 / BATCH)
    inv = 1.0 / jnp.sqrt(ex2 - m * m + 1e-5)
    h = (y_ref[...] - m) * (inv * gm_ref[...]) + bb_ref[...]
    y2 = jnp.dot(h.astype(jnp.bfloat16), w_ref[...].astype(jnp.bfloat16),
                 preferred_element_type=jnp.float32) + b_ref[...]
    y2_ref[...] = y2
    s2_ref[...] = jnp.sum(y2, axis=0, keepdims=True)[None]
    q2_ref[...] = jnp.sum(y2 * y2, axis=0, keepdims=True)[None]


def _mid_layer(y, s, q, w, b, gm, bb):
    din, dout = w.shape
    return pl.pallas_call(
        _mid_body,
        grid=(BT,),
        in_specs=[
            pl.BlockSpec((BM, din), lambda i: (i, 0)),
            pl.BlockSpec((BT, 1, din), lambda i: (0, 0, 0)),
            pl.BlockSpec((BT, 1, din), lambda i: (0, 0, 0)),
            pl.BlockSpec((din, dout), lambda i: (0, 0)),
            pl.BlockSpec((1, dout), lambda i: (0, 0)),
            pl.BlockSpec((1, din), lambda i: (0, 0)),
            pl.BlockSpec((1, din), lambda i: (0, 0)),
        ],
        out_specs=[
            pl.BlockSpec((BM, dout), lambda i: (i, 0)),
            pl.BlockSpec((1, 1, dout), lambda i: (i, 0, 0)),
            pl.BlockSpec((1, 1, dout), lambda i: (i, 0, 0)),
        ],
        out_shape=[
            jax.ShapeDtypeStruct((BATCH, dout), jnp.float32),
            jax.ShapeDtypeStruct((BT, 1, dout), jnp.float32),
            jax.ShapeDtypeStruct((BT, 1, dout), jnp.float32),
        ],
        compiler_params=pltpu.CompilerParams(
            dimension_semantics=("arbitrary",)),
    )(y, s, q, w, b, gm, bb)


def _fin_body(y_ref, s_ref, q_ref, w_ref, b_ref, gm_ref, bb_ref, o_ref):
    m = jnp.sum(s_ref[...], axis=0) * (1.0 / BATCH)
    ex2 = jnp.sum(q_ref[...], axis=0) * (1.0 / BATCH)
    inv = 1.0 / jnp.sqrt(ex2 - m * m + 1e-5)
    h = (y_ref[...] - m) * (inv * gm_ref[...]) + bb_ref[...]
    o_ref[...] = jax.nn.sigmoid(
        jnp.dot(h, w_ref[...], preferred_element_type=jnp.float32) + b_ref[...])


def _fin_layer(y, s, q, w, b, gm, bb):
    din, dout = w.shape
    return pl.pallas_call(
        _fin_body,
        grid=(BT,),
        in_specs=[
            pl.BlockSpec((BM, din), lambda i: (i, 0)),
            pl.BlockSpec((BT, 1, din), lambda i: (0, 0, 0)),
            pl.BlockSpec((BT, 1, din), lambda i: (0, 0, 0)),
            pl.BlockSpec((din, dout), lambda i: (0, 0)),
            pl.BlockSpec((1, dout), lambda i: (0, 0)),
            pl.BlockSpec((1, din), lambda i: (0, 0)),
            pl.BlockSpec((1, din), lambda i: (0, 0)),
        ],
        out_specs=pl.BlockSpec((BM, dout), lambda i: (i, 0)),
        out_shape=jax.ShapeDtypeStruct((BATCH, dout), jnp.float32),
        compiler_params=pltpu.CompilerParams(
            dimension_semantics=("arbitrary",)),
    )(y, s, q, w, b, gm, bb)


def kernel(x, emb1, emb2_first, emb2_second, emb3, title_table,
           Ws, bs, bn_scales, bn_biases):
    x = x.astype(jnp.int32)
    xt = x[:, :28].T
    xd = x[:, 28:284]
    g = _sc_gather(xt, *emb1, *emb2_first, *emb2_second, *emb3, title_table)
    g3 = g.reshape(NPLANE, BATCH, EMB)
    we = Ws[0][:NPLANE * EMB].reshape(NPLANE, EMB, D1)
    wd = Ws[0][NPLANE * EMB:]
    y1, s1, q1 = _layer1(g3, xd, we, wd, bs[0].reshape(1, -1))
    y2, s2, q2 = _mid_layer(y1, s1, q1, Ws[1], bs[1].reshape(1, -1),
                            bn_scales[0].reshape(1, -1),
                            bn_biases[0].reshape(1, -1))
    y3, s3, q3 = _mid_layer(y2, s2, q2, Ws[2], bs[2].reshape(1, -1),
                            bn_scales[1].reshape(1, -1),
                            bn_biases[1].reshape(1, -1))
    w4 = jnp.pad(Ws[3], ((0, 0), (0, EMB - Ws[3].shape[1])))
    b4 = jnp.pad(bs[3], (0, EMB - bs[3].shape[0])).reshape(1, -1)
    o = _fin_layer(y3, s3, q3, w4, b4,
                   bn_scales[2].reshape(1, -1), bn_biases[2].reshape(1, -1))
    return o[:, :1]
